# Initial kernel scaffold; baseline (speedup 1.0000x reference)
import jax
import jax.numpy as jnp
from jax.experimental import pallas as pl


def kernel(x_s, fc1_W, fc1_b, Wh_W, Wh_b, Wt_W, Wt_b, l1_W, l1_b, l2_W, l2_b, att1_W, att1_b, att2_W, att2_b, ln_g, ln_b, fc_W, fc_b):
    raise NotImplementedError("write your pallas kernel here")


# speedup vs baseline: 10.6825x; 10.6825x over previous
import functools

import jax
import jax.numpy as jnp
from jax import lax
from jax.experimental import pallas as pl
from jax.experimental.pallas import tpu as pltpu
from jax.experimental.pallas import tpu_sc as plsc

_N = 4096
_DIN = 384
_DH = 512
_K = 6
_KP = 8
_NCLS = 2
_SCALE = _DH ** (-0.5)
_BR = 256
_GRID = _N // _BR


def _fc1_body(xs_ref, w_ref, b_ref, x0_ref, cs_ref):
    y = jnp.dot(xs_ref[...], w_ref[...], preferred_element_type=jnp.float32)
    y = y + b_ref[...]
    y = jnp.where(y >= 0, y, 0.01 * y)
    x0_ref[...] = y
    s = jnp.sum(y, axis=0, keepdims=True)

    @pl.when(pl.program_id(0) == 0)
    def _():
        cs_ref[...] = s

    @pl.when(pl.program_id(0) != 0)
    def _():
        cs_ref[...] = cs_ref[...] + s


def _fc1_stage(x_s, fc1_W, fc1_b2):
    return pl.pallas_call(
        _fc1_body,
        grid=(_GRID,),
        in_specs=[
            pl.BlockSpec((_BR, _DIN), lambda i: (i, 0)),
            pl.BlockSpec((_DIN, _DH), lambda i: (0, 0)),
            pl.BlockSpec((1, _DH), lambda i: (0, 0)),
        ],
        out_specs=[
            pl.BlockSpec((_BR, _DH), lambda i: (i, 0)),
            pl.BlockSpec((1, _DH), lambda i: (0, 0)),
        ],
        out_shape=[
            jax.ShapeDtypeStruct((_N, _DH), jnp.float32),
            jax.ShapeDtypeStruct((1, _DH), jnp.float32),
        ],
    )(x_s, fc1_W, fc1_b2)


def _proj_body(x0_ref, cs_ref, wh_ref, bh_ref, wt_ref, bt_ref, eh_ref, et_ref):
    x = (x0_ref[...] + cs_ref[...] * (1.0 / _N)) * 0.5
    eh_ref[...] = jnp.dot(x, wh_ref[...], preferred_element_type=jnp.float32) + bh_ref[...]
    et_ref[...] = jnp.dot(x, wt_ref[...], preferred_element_type=jnp.float32) + bt_ref[...]


def _proj_stage(x0, csum, Wh_W, bh2, Wt_W, bt2):
    return pl.pallas_call(
        _proj_body,
        grid=(_GRID,),
        in_specs=[
            pl.BlockSpec((_BR, _DH), lambda i: (i, 0)),
            pl.BlockSpec((1, _DH), lambda i: (0, 0)),
            pl.BlockSpec((_DH, _DH), lambda i: (0, 0)),
            pl.BlockSpec((1, _DH), lambda i: (0, 0)),
            pl.BlockSpec((_DH, _DH), lambda i: (0, 0)),
            pl.BlockSpec((1, _DH), lambda i: (0, 0)),
        ],
        out_specs=[
            pl.BlockSpec((_BR, _DH), lambda i: (i, 0)),
            pl.BlockSpec((_BR, _DH), lambda i: (i, 0)),
        ],
        out_shape=[
            jax.ShapeDtypeStruct((_N, _DH), jnp.float32),
            jax.ShapeDtypeStruct((_N, _DH), jnp.float32),
        ],
    )(x0, csum, Wh_W, bh2, Wt_W, bt2)


def _topk_body(eh_ref, et_ref, p_ref, idx_ref):
    eh = eh_ref[...] * _SCALE
    logits = lax.dot_general(
        eh, et_ref[...], (((1,), (1,)), ((), ())),
        preferred_element_type=jnp.float32)
    col = lax.broadcasted_iota(jnp.int32, (_BR, _N), 1)
    ws = []
    for k in range(_K):
        m = jnp.max(logits, axis=1)
        ik = jnp.min(jnp.where(logits == m[:, None], col, _N), axis=1)
        idx_ref[k:k + 1, :] = ik[None, :]
        ws.append(m)
        logits = jnp.where(col == ik[:, None], -jnp.inf, logits)
    mm = ws[0]
    es = [jnp.exp(w - mm) for w in ws]
    tot = es[0]
    for e in es[1:]:
        tot = tot + e
    for k in range(_K):
        p_ref[:, k:k + 1] = (es[k] / tot)[:, None]


def _topk_stage(e_h, e_t):
    return pl.pallas_call(
        _topk_body,
        grid=(_GRID,),
        in_specs=[
            pl.BlockSpec((_BR, _DH), lambda i: (i, 0)),
            pl.BlockSpec((_N, _DH), lambda i: (0, 0)),
        ],
        out_specs=[
            pl.BlockSpec((_BR, _KP), lambda i: (i, 0)),
            pl.BlockSpec((_KP, _BR), lambda i: (0, i)),
        ],
        out_shape=[
            jax.ShapeDtypeStruct((_N, _KP), jnp.float32),
            jax.ShapeDtypeStruct((_KP, _N), jnp.int32),
        ],
    )(e_h, e_t)


_NW = 32
_B_GATHER = _K * _N
_BPW = _B_GATHER // _NW
_CH = 64
_NCH = _BPW // _CH


def _sc_gather_body(table_hbm, idx_hbm, out_hbm, idx_v, buf0, buf1,
                    g0, g1, s0, s1):
    wid = lax.axis_index("s") * 2 + lax.axis_index("c")
    base = wid * _BPW
    pltpu.sync_copy(idx_hbm.at[pl.ds(base, _BPW)], idx_v)
    bufs = (buf0, buf1)
    gsems = (g0, g1)
    ssems = (s0, s1)
    pending = [None, None]
    for c in range(_NCH):
        b = c & 1
        if pending[b] is not None:
            pending[b].wait()
        pltpu.async_copy(
            table_hbm.at[idx_v.at[pl.ds(c * _CH, _CH)]], bufs[b],
            gsems[b]).wait()
        d = pltpu.make_async_copy(
            bufs[b], out_hbm.at[pl.ds(base + c * _CH, _CH)], ssems[b])
        d.start()
        pending[b] = d
    for b in range(2):
        if pending[b] is not None:
            pending[b].wait()


def _sc_gather(table, idx_flat):
    mesh = plsc.VectorSubcoreMesh(core_axis_name="c", subcore_axis_name="s")
    fn = functools.partial(
        pl.kernel,
        mesh=mesh,
        out_type=jax.ShapeDtypeStruct((_B_GATHER, _DH), jnp.float32),
        scratch_types=[
            pltpu.VMEM((_BPW,), jnp.int32),
            pltpu.VMEM((_CH, _DH), jnp.float32),
            pltpu.VMEM((_CH, _DH), jnp.float32),
            pltpu.SemaphoreType.DMA,
            pltpu.SemaphoreType.DMA,
            pltpu.SemaphoreType.DMA,
            pltpu.SemaphoreType.DMA,
        ],
    )(_sc_gather_body)
    return fn(table, idx_flat)


def _agg_body(eh_ref, p_ref, nb_ref, l1w_ref, l1b_ref, l2w_ref, l2b_ref,
              a1w_ref, a1b_ref, a2w_ref, a2b_ref, h_ref, gs_ref):
    eh = eh_ref[...]
    kas = []
    for k in range(_K):
        nbk = nb_ref[k]
        pk = p_ref[:, k:k + 1]
        ehr = pk * nbk + (1.0 - pk) * eh
        gate = jnp.tanh(eh + ehr)
        ka = (jnp.sum(nbk, axis=1, keepdims=True)
              * jnp.sum(gate, axis=1, keepdims=True))
        kas.append(ka)
    mka = kas[0]
    for ka in kas[1:]:
        mka = jnp.maximum(mka, ka)
    eks = [jnp.exp(ka - mka) for ka in kas]
    tot = eks[0]
    for e in eks[1:]:
        tot = tot + e
    e_nh = (eks[0] / tot) * nb_ref[0]
    for k in range(1, _K):
        e_nh = e_nh + (eks[k] / tot) * nb_ref[k]
    sum_emb = jnp.dot(eh + e_nh, l1w_ref[...],
                      preferred_element_type=jnp.float32) + l1b_ref[...]
    sum_emb = jnp.where(sum_emb >= 0, sum_emb, 0.01 * sum_emb)
    bi_emb = jnp.dot(eh * e_nh, l2w_ref[...],
                     preferred_element_type=jnp.float32) + l2b_ref[...]
    bi_emb = jnp.where(bi_emb >= 0, bi_emb, 0.01 * bi_emb)
    h = sum_emb + bi_emb
    h_ref[...] = h
    g1 = jnp.dot(h, a1w_ref[...], preferred_element_type=jnp.float32) + a1b_ref[...]
    g1 = jnp.where(g1 >= 0, g1, 0.01 * g1)
    gs = jnp.sum(g1 * a2w_ref[...], axis=1, keepdims=True) + a2b_ref[...]
    gs_ref[:, 0:1] = gs


def _agg_stage(e_h, p, nb3, l1_W, l1b2, l2_W, l2b2, att1_W, a1b2, a2wr, a2b2):
    return pl.pallas_call(
        _agg_body,
        grid=(_GRID,),
        in_specs=[
            pl.BlockSpec((_BR, _DH), lambda i: (i, 0)),
            pl.BlockSpec((_BR, _KP), lambda i: (i, 0)),
            pl.BlockSpec((_K, _BR, _DH), lambda i: (0, i, 0)),
            pl.BlockSpec((_DH, _DH), lambda i: (0, 0)),
            pl.BlockSpec((1, _DH), lambda i: (0, 0)),
            pl.BlockSpec((_DH, _DH), lambda i: (0, 0)),
            pl.BlockSpec((1, _DH), lambda i: (0, 0)),
            pl.BlockSpec((_DH, _DH // 2), lambda i: (0, 0)),
            pl.BlockSpec((1, _DH // 2), lambda i: (0, 0)),
            pl.BlockSpec((1, _DH // 2), lambda i: (0, 0)),
            pl.BlockSpec((1, 1), lambda i: (0, 0)),
        ],
        out_specs=[
            pl.BlockSpec((_BR, _DH), lambda i: (i, 0)),
            pl.BlockSpec((_BR, _KP), lambda i: (i, 0)),
        ],
        out_shape=[
            jax.ShapeDtypeStruct((_N, _DH), jnp.float32),
            jax.ShapeDtypeStruct((_N, _KP), jnp.float32),
        ],
    )(e_h, p, nb3, l1_W, l1b2, l2_W, l2b2, att1_W, a1b2, a2wr, a2b2)


def _readout_body(h_ref, gs_ref, g_ref, b_ref, fcwt_ref, fcb_ref,
                  lg_ref, yp_ref):
    gs = gs_ref[...][:, 0:1]
    m = jnp.max(gs, axis=0, keepdims=True)
    e = jnp.exp(gs - m)
    alpha = e / jnp.sum(e, axis=0, keepdims=True)
    hp = jnp.sum(alpha * h_ref[...], axis=0, keepdims=True)
    mu = jnp.mean(hp, axis=1, keepdims=True)
    var = jnp.mean((hp - mu) ** 2, axis=1, keepdims=True)
    hn = (hp - mu) / jnp.sqrt(var + 1e-5) * g_ref[...] + b_ref[...]
    lgs = []
    for c in range(_NCLS):
        v = jnp.sum(hn * fcwt_ref[c:c + 1, :], axis=1, keepdims=True)
        lgs.append(v + fcb_ref[:, c:c + 1])
    m2 = jnp.maximum(lgs[0], lgs[1])
    e2 = [jnp.exp(v - m2) for v in lgs]
    s2 = e2[0] + e2[1]
    for c in range(_NCLS):
        lg_ref[:, c:c + 1] = lgs[c]
        yp_ref[:, c:c + 1] = e2[c] / s2


def _readout_stage(h, gs, ln_g2, ln_b2, fc_WT, fc_b2):
    return pl.pallas_call(
        _readout_body,
        out_shape=[
            jax.ShapeDtypeStruct((1, _NCLS), jnp.float32),
            jax.ShapeDtypeStruct((1, _NCLS), jnp.float32),
        ],
    )(h, gs, ln_g2, ln_b2, fc_WT, fc_b2)


def kernel(x_s, fc1_W, fc1_b, Wh_W, Wh_b, Wt_W, Wt_b, l1_W, l1_b, l2_W, l2_b,
           att1_W, att1_b, att2_W, att2_b, ln_g, ln_b, fc_W, fc_b):
    fc1_b2 = fc1_b.reshape(1, _DH)
    bh2 = Wh_b.reshape(1, _DH)
    bt2 = Wt_b.reshape(1, _DH)
    l1b2 = l1_b.reshape(1, _DH)
    l2b2 = l2_b.reshape(1, _DH)
    a1b2 = att1_b.reshape(1, _DH // 2)
    a2wr = att2_W.reshape(1, _DH // 2)
    a2b2 = att2_b.reshape(1, 1)
    ln_g2 = ln_g.reshape(1, _DH)
    ln_b2 = ln_b.reshape(1, _DH)
    fc_WT = fc_W.T
    fc_b2 = fc_b.reshape(1, _NCLS)

    x0, csum = _fc1_stage(x_s, fc1_W, fc1_b2)
    e_h, e_t = _proj_stage(x0, csum, Wh_W, bh2, Wt_W, bt2)
    p, idx = _topk_stage(e_h, e_t)
    idx_flat = idx[:_K].reshape(_K * _N)
    nb = _sc_gather(e_t, idx_flat)
    nb3 = nb.reshape(_K, _N, _DH)
    h, gs = _agg_stage(e_h, p, nb3, l1_W, l1b2, l2_W, l2b2,
                       att1_W, a1b2, a2wr, a2b2)
    logits, y_prob = _readout_stage(h, gs, ln_g2, ln_b2, fc_WT, fc_b2)
    return logits, y_prob


# speedup vs baseline: 12.5247x; 1.1725x over previous
import functools

import jax
import jax.numpy as jnp
from jax import lax
from jax.experimental import pallas as pl
from jax.experimental.pallas import tpu as pltpu
from jax.experimental.pallas import tpu_sc as plsc

_N = 4096
_DIN = 384
_DH = 512
_K = 6
_KP = 8
_NCLS = 2
_SCALE = _DH ** (-0.5)
_BR = 256
_NSPLIT = 2


def _fc1_body(xs_ref, w_ref, b_ref, x0_ref, cs_ref):
    y = jnp.dot(xs_ref[...], w_ref[...], preferred_element_type=jnp.float32)
    y = y + b_ref[...]
    y = jnp.where(y >= 0, y, 0.01 * y)
    x0_ref[...] = y
    s = jnp.sum(y, axis=0, keepdims=True)

    @pl.when(pl.program_id(0) == 0)
    def _():
        cs_ref[...] = s

    @pl.when(pl.program_id(0) != 0)
    def _():
        cs_ref[...] = cs_ref[...] + s


def _fc1_stage(x_s, fc1_W, fc1_b2):
    return pl.pallas_call(
        _fc1_body,
        grid=(_N // _BR,),
        in_specs=[
            pl.BlockSpec((_BR, _DIN), lambda i: (i, 0)),
            pl.BlockSpec((_DIN, _DH), lambda i: (0, 0)),
            pl.BlockSpec((1, _DH), lambda i: (0, 0)),
        ],
        out_specs=[
            pl.BlockSpec((_BR, _DH), lambda i: (i, 0)),
            pl.BlockSpec((1, _DH), lambda i: (0, 0)),
        ],
        out_shape=[
            jax.ShapeDtypeStruct((_N, _DH), jnp.float32),
            jax.ShapeDtypeStruct((1, _DH), jnp.float32),
        ],
    )(x_s, fc1_W, fc1_b2)


def _proj_body(x0_ref, cs_ref, wh_ref, bh_ref, wt_ref, bt_ref, eh_ref, et_ref):
    x = (x0_ref[...] + cs_ref[...] * (1.0 / _N)) * 0.5
    eh_ref[...] = jnp.dot(x, wh_ref[...], preferred_element_type=jnp.float32) + bh_ref[...]
    et_ref[...] = jnp.dot(x, wt_ref[...], preferred_element_type=jnp.float32) + bt_ref[...]


def _proj_stage(x0, csum, Wh_W, bh2, Wt_W, bt2):
    return pl.pallas_call(
        _proj_body,
        grid=(_N // _BR,),
        in_specs=[
            pl.BlockSpec((_BR, _DH), lambda i: (i, 0)),
            pl.BlockSpec((1, _DH), lambda i: (0, 0)),
            pl.BlockSpec((_DH, _DH), lambda i: (0, 0)),
            pl.BlockSpec((1, _DH), lambda i: (0, 0)),
            pl.BlockSpec((_DH, _DH), lambda i: (0, 0)),
            pl.BlockSpec((1, _DH), lambda i: (0, 0)),
        ],
        out_specs=[
            pl.BlockSpec((_BR, _DH), lambda i: (i, 0)),
            pl.BlockSpec((_BR, _DH), lambda i: (i, 0)),
        ],
        out_shape=[
            jax.ShapeDtypeStruct((_N, _DH), jnp.float32),
            jax.ShapeDtypeStruct((_N, _DH), jnp.float32),
        ],
    )(x0, csum, Wh_W, bh2, Wt_W, bt2)


def _topk_body(eh_ref, et_ref, p_ref, idx_ref):
    eh = eh_ref[...] * _SCALE
    logits = lax.dot_general(
        eh, et_ref[...], (((1,), (1,)), ((), ())),
        preferred_element_type=jnp.float32)
    col = lax.broadcasted_iota(jnp.int32, (_BR, _N), 1)
    ws = []
    for k in range(_K):
        m = jnp.max(logits, axis=1)
        ik = jnp.min(jnp.where(logits == m[:, None], col, _N), axis=1)
        idx_ref[k:k + 1, :] = ik[None, :]
        ws.append(m)
        logits = jnp.where(col == ik[:, None], -jnp.inf, logits)
    mm = ws[0]
    es = [jnp.exp(w - mm) for w in ws]
    tot = es[0]
    for e in es[1:]:
        tot = tot + e
    for k in range(_K):
        p_ref[:, k:k + 1] = (es[k] / tot)[:, None]


def _topk_stage(e_h_c, e_t):
    rows = e_h_c.shape[0]
    return pl.pallas_call(
        _topk_body,
        grid=(rows // _BR,),
        in_specs=[
            pl.BlockSpec((_BR, _DH), lambda i: (i, 0)),
            pl.BlockSpec((_N, _DH), lambda i: (0, 0)),
        ],
        out_specs=[
            pl.BlockSpec((_BR, _KP), lambda i: (i, 0)),
            pl.BlockSpec((_KP, _BR), lambda i: (0, i)),
        ],
        out_shape=[
            jax.ShapeDtypeStruct((rows, _KP), jnp.float32),
            jax.ShapeDtypeStruct((_KP, rows), jnp.int32),
        ],
    )(e_h_c, e_t)


_NW = 32
_CH = 64
_NBUF = 3


def _sc_gather_body(nch, table_hbm, idx_hbm, out_hbm, idx_v, buf0, buf1, buf2,
                    g0, g1, g2, s0, s1, s2):
    bpw = nch * _CH
    wid = lax.axis_index("s") * 2 + lax.axis_index("c")
    base = wid * bpw
    pltpu.sync_copy(idx_hbm.at[pl.ds(base, bpw)], idx_v)
    bufs = (buf0, buf1, buf2)
    gsems = (g0, g1, g2)
    ssems = (s0, s1, s2)
    gd = [None] * _NBUF
    sd = [None] * _NBUF

    def start_gather(c):
        b = c % _NBUF
        if sd[b] is not None:
            sd[b].wait()
            sd[b] = None
        d = pltpu.make_async_copy(
            table_hbm.at[idx_v.at[pl.ds(c * _CH, _CH)]], bufs[b], gsems[b])
        d.start()
        gd[b] = d

    start_gather(0)
    if nch > 1:
        start_gather(1)
    for c in range(nch):
        b = c % _NBUF
        gd[b].wait()
        d = pltpu.make_async_copy(
            bufs[b], out_hbm.at[pl.ds(base + c * _CH, _CH)], ssems[b])
        d.start()
        sd[b] = d
        if c + 2 < nch:
            start_gather(c + 2)
    for b in range(_NBUF):
        if sd[b] is not None:
            sd[b].wait()


def _sc_gather(table, idx_flat):
    nrows = idx_flat.shape[0]
    bpw = nrows // _NW
    nch = bpw // _CH
    mesh = plsc.VectorSubcoreMesh(core_axis_name="c", subcore_axis_name="s")
    fn = functools.partial(
        pl.kernel,
        mesh=mesh,
        out_type=jax.ShapeDtypeStruct((nrows, _DH), jnp.float32),
        scratch_types=[
            pltpu.VMEM((bpw,), jnp.int32),
            pltpu.VMEM((_CH, _DH), jnp.float32),
            pltpu.VMEM((_CH, _DH), jnp.float32),
            pltpu.VMEM((_CH, _DH), jnp.float32),
            pltpu.SemaphoreType.DMA,
            pltpu.SemaphoreType.DMA,
            pltpu.SemaphoreType.DMA,
            pltpu.SemaphoreType.DMA,
            pltpu.SemaphoreType.DMA,
            pltpu.SemaphoreType.DMA,
        ],
    )(functools.partial(_sc_gather_body, nch))
    return fn(table, idx_flat)


def _agg_body(eh_ref, p_ref, nb_ref, l1w_ref, l1b_ref, l2w_ref, l2b_ref,
              a1w_ref, a1b_ref, a2w_ref, a2b_ref, h_ref, gs_ref):
    eh = eh_ref[...]
    kas = []
    for k in range(_K):
        nbk = nb_ref[k]
        pk = p_ref[:, k:k + 1]
        ehr = pk * nbk + (1.0 - pk) * eh
        gate = jnp.tanh(eh + ehr)
        ka = (jnp.sum(nbk, axis=1, keepdims=True)
              * jnp.sum(gate, axis=1, keepdims=True))
        kas.append(ka)
    mka = kas[0]
    for ka in kas[1:]:
        mka = jnp.maximum(mka, ka)
    eks = [jnp.exp(ka - mka) for ka in kas]
    tot = eks[0]
    for e in eks[1:]:
        tot = tot + e
    e_nh = (eks[0] / tot) * nb_ref[0]
    for k in range(1, _K):
        e_nh = e_nh + (eks[k] / tot) * nb_ref[k]
    sum_emb = jnp.dot(eh + e_nh, l1w_ref[...],
                      preferred_element_type=jnp.float32) + l1b_ref[...]
    sum_emb = jnp.where(sum_emb >= 0, sum_emb, 0.01 * sum_emb)
    bi_emb = jnp.dot(eh * e_nh, l2w_ref[...],
                     preferred_element_type=jnp.float32) + l2b_ref[...]
    bi_emb = jnp.where(bi_emb >= 0, bi_emb, 0.01 * bi_emb)
    h = sum_emb + bi_emb
    h_ref[...] = h
    g1 = jnp.dot(h, a1w_ref[...], preferred_element_type=jnp.float32) + a1b_ref[...]
    g1 = jnp.where(g1 >= 0, g1, 0.01 * g1)
    gs = jnp.sum(g1 * a2w_ref[...], axis=1, keepdims=True) + a2b_ref[...]
    gs_ref[:, 0:1] = gs


def _agg_stage(e_h_c, p, nb3, l1_W, l1b2, l2_W, l2b2, att1_W, a1b2, a2wr, a2b2):
    rows = e_h_c.shape[0]
    return pl.pallas_call(
        _agg_body,
        grid=(rows // _BR,),
        in_specs=[
            pl.BlockSpec((_BR, _DH), lambda i: (i, 0)),
            pl.BlockSpec((_BR, _KP), lambda i: (i, 0)),
            pl.BlockSpec((_K, _BR, _DH), lambda i: (0, i, 0)),
            pl.BlockSpec((_DH, _DH), lambda i: (0, 0)),
            pl.BlockSpec((1, _DH), lambda i: (0, 0)),
            pl.BlockSpec((_DH, _DH), lambda i: (0, 0)),
            pl.BlockSpec((1, _DH), lambda i: (0, 0)),
            pl.BlockSpec((_DH, _DH // 2), lambda i: (0, 0)),
            pl.BlockSpec((1, _DH // 2), lambda i: (0, 0)),
            pl.BlockSpec((1, _DH // 2), lambda i: (0, 0)),
            pl.BlockSpec((1, 1), lambda i: (0, 0)),
        ],
        out_specs=[
            pl.BlockSpec((_BR, _DH), lambda i: (i, 0)),
            pl.BlockSpec((_BR, _KP), lambda i: (i, 0)),
        ],
        out_shape=[
            jax.ShapeDtypeStruct((rows, _DH), jnp.float32),
            jax.ShapeDtypeStruct((rows, _KP), jnp.float32),
        ],
    )(e_h_c, p, nb3, l1_W, l1b2, l2_W, l2b2, att1_W, a1b2, a2wr, a2b2)


def _readout_body(g_ref, b_ref, fcwt_ref, fcb_ref, *rest):
    h_refs = rest[:_NSPLIT]
    gs_refs = rest[_NSPLIT:2 * _NSPLIT]
    lg_ref, yp_ref = rest[2 * _NSPLIT], rest[2 * _NSPLIT + 1]
    gss = [r[...][:, 0:1] for r in gs_refs]
    m = jnp.max(gss[0], axis=0, keepdims=True)
    for g in gss[1:]:
        m = jnp.maximum(m, jnp.max(g, axis=0, keepdims=True))
    es = [jnp.exp(g - m) for g in gss]
    tot = jnp.sum(es[0], axis=0, keepdims=True)
    for e in es[1:]:
        tot = tot + jnp.sum(e, axis=0, keepdims=True)
    hp = jnp.sum(es[0] * h_refs[0][...], axis=0, keepdims=True)
    for e, hr in zip(es[1:], h_refs[1:]):
        hp = hp + jnp.sum(e * hr[...], axis=0, keepdims=True)
    hp = hp / tot
    mu = jnp.mean(hp, axis=1, keepdims=True)
    var = jnp.mean((hp - mu) ** 2, axis=1, keepdims=True)
    hn = (hp - mu) / jnp.sqrt(var + 1e-5) * g_ref[...] + b_ref[...]
    lgs = []
    for c in range(_NCLS):
        v = jnp.sum(hn * fcwt_ref[c:c + 1, :], axis=1, keepdims=True)
        lgs.append(v + fcb_ref[:, c:c + 1])
    m2 = jnp.maximum(lgs[0], lgs[1])
    e2 = [jnp.exp(v - m2) for v in lgs]
    s2 = e2[0] + e2[1]
    for c in range(_NCLS):
        lg_ref[:, c:c + 1] = lgs[c]
        yp_ref[:, c:c + 1] = e2[c] / s2


def _readout_stage(hs, gss, ln_g2, ln_b2, fc_WT, fc_b2):
    return pl.pallas_call(
        _readout_body,
        out_shape=[
            jax.ShapeDtypeStruct((1, _NCLS), jnp.float32),
            jax.ShapeDtypeStruct((1, _NCLS), jnp.float32),
        ],
    )(ln_g2, ln_b2, fc_WT, fc_b2, *hs, *gss)


def kernel(x_s, fc1_W, fc1_b, Wh_W, Wh_b, Wt_W, Wt_b, l1_W, l1_b, l2_W, l2_b,
           att1_W, att1_b, att2_W, att2_b, ln_g, ln_b, fc_W, fc_b):
    fc1_b2 = fc1_b.reshape(1, _DH)
    bh2 = Wh_b.reshape(1, _DH)
    bt2 = Wt_b.reshape(1, _DH)
    l1b2 = l1_b.reshape(1, _DH)
    l2b2 = l2_b.reshape(1, _DH)
    a1b2 = att1_b.reshape(1, _DH // 2)
    a2wr = att2_W.reshape(1, _DH // 2)
    a2b2 = att2_b.reshape(1, 1)
    ln_g2 = ln_g.reshape(1, _DH)
    ln_b2 = ln_b.reshape(1, _DH)
    fc_WT = fc_W.T
    fc_b2 = fc_b.reshape(1, _NCLS)

    x0, csum = _fc1_stage(x_s, fc1_W, fc1_b2)
    e_h, e_t = _proj_stage(x0, csum, Wh_W, bh2, Wt_W, bt2)

    rows = _N // _NSPLIT
    ehcs, pcs, nbcs = [], [], []
    for c in range(_NSPLIT):
        ehc = lax.slice(e_h, (c * rows, 0), ((c + 1) * rows, _DH))
        p_c, idx_c = _topk_stage(ehc, e_t)
        idx_flat = idx_c[:_K].reshape(_K * rows)
        nb_c = _sc_gather(e_t, idx_flat)
        ehcs.append(ehc)
        pcs.append(p_c)
        nbcs.append(nb_c.reshape(_K, rows, _DH))
    hs, gss = [], []
    for c in range(_NSPLIT):
        h_c, gs_c = _agg_stage(ehcs[c], pcs[c], nbcs[c], l1_W, l1b2, l2_W,
                               l2b2, att1_W, a1b2, a2wr, a2b2)
        hs.append(h_c)
        gss.append(gs_c)
    logits, y_prob = _readout_stage(hs, gss, ln_g2, ln_b2, fc_WT, fc_b2)
    return logits, y_prob


# speedup vs baseline: 13.1889x; 1.0530x over previous
import functools

import jax
import jax.numpy as jnp
from jax import lax
from jax.experimental import pallas as pl
from jax.experimental.pallas import tpu as pltpu
from jax.experimental.pallas import tpu_sc as plsc

_N = 4096
_DIN = 384
_DH = 512
_K = 6
_KP = 8
_NCLS = 2
_SCALE = _DH ** (-0.5)
_BR = 256
_NSPLIT = 2


def _fc1_body(xs_ref, w_ref, b_ref, x0_ref, cs_ref):
    y = jnp.dot(xs_ref[...], w_ref[...], preferred_element_type=jnp.float32)
    y = y + b_ref[...]
    y = jnp.where(y >= 0, y, 0.01 * y)
    x0_ref[...] = y
    s = jnp.sum(y, axis=0, keepdims=True)

    @pl.when(pl.program_id(0) == 0)
    def _():
        cs_ref[...] = s

    @pl.when(pl.program_id(0) != 0)
    def _():
        cs_ref[...] = cs_ref[...] + s


def _fc1_stage(x_s, fc1_W, fc1_b2):
    return pl.pallas_call(
        _fc1_body,
        grid=(_N // _BR,),
        in_specs=[
            pl.BlockSpec((_BR, _DIN), lambda i: (i, 0)),
            pl.BlockSpec((_DIN, _DH), lambda i: (0, 0)),
            pl.BlockSpec((1, _DH), lambda i: (0, 0)),
        ],
        out_specs=[
            pl.BlockSpec((_BR, _DH), lambda i: (i, 0)),
            pl.BlockSpec((1, _DH), lambda i: (0, 0)),
        ],
        out_shape=[
            jax.ShapeDtypeStruct((_N, _DH), jnp.float32),
            jax.ShapeDtypeStruct((1, _DH), jnp.float32),
        ],
    )(x_s, fc1_W, fc1_b2)


def _proj_body(x0_ref, cs_ref, wh_ref, bh_ref, wt_ref, bt_ref, eh_ref, et_ref):
    x = (x0_ref[...] + cs_ref[...] * (1.0 / _N)) * 0.5
    eh_ref[...] = jnp.dot(x, wh_ref[...], preferred_element_type=jnp.float32) + bh_ref[...]
    et_ref[...] = jnp.dot(x, wt_ref[...], preferred_element_type=jnp.float32) + bt_ref[...]


def _proj_stage(x0, csum, Wh_W, bh2, Wt_W, bt2):
    return pl.pallas_call(
        _proj_body,
        grid=(_N // _BR,),
        in_specs=[
            pl.BlockSpec((_BR, _DH), lambda i: (i, 0)),
            pl.BlockSpec((1, _DH), lambda i: (0, 0)),
            pl.BlockSpec((_DH, _DH), lambda i: (0, 0)),
            pl.BlockSpec((1, _DH), lambda i: (0, 0)),
            pl.BlockSpec((_DH, _DH), lambda i: (0, 0)),
            pl.BlockSpec((1, _DH), lambda i: (0, 0)),
        ],
        out_specs=[
            pl.BlockSpec((_BR, _DH), lambda i: (i, 0)),
            pl.BlockSpec((_BR, _DH), lambda i: (i, 0)),
        ],
        out_shape=[
            jax.ShapeDtypeStruct((_N, _DH), jnp.float32),
            jax.ShapeDtypeStruct((_N, _DH), jnp.float32),
        ],
    )(x0, csum, Wh_W, bh2, Wt_W, bt2)


def _topk_body(eh_ref, et_ref, p_ref, idx_ref):
    eh = eh_ref[...] * _SCALE
    logits = lax.dot_general(
        eh, et_ref[...], (((1,), (1,)), ((), ())),
        preferred_element_type=jnp.float32)
    imin = jnp.int32(-2147483648)
    ib = lax.bitcast_convert_type(logits, jnp.int32)
    ks = jnp.where(ib >= 0, ib, jnp.bitwise_xor(jnp.bitwise_not(ib), imin))
    colrev = (jnp.int32(_N - 1)
              - lax.broadcasted_iota(jnp.int32, (_BR, _N), 1))
    packed = jnp.bitwise_or(jnp.bitwise_and(ks, jnp.int32(-4096)), colrev)
    ws = []
    for k in range(_K):
        m = jnp.max(packed, axis=1)
        ik = jnp.int32(_N - 1) - jnp.bitwise_and(m, jnp.int32(_N - 1))
        idx_ref[k:k + 1, :] = ik[None, :]
        kh = jnp.bitwise_and(m, jnp.int32(-4096))
        ib2 = jnp.where(kh >= 0, kh,
                        jnp.bitwise_not(jnp.bitwise_xor(kh, imin)))
        ws.append(lax.bitcast_convert_type(ib2, jnp.float32))
        packed = jnp.where(packed == m[:, None], imin, packed)
    mm = ws[0]
    es = [jnp.exp(w - mm) for w in ws]
    tot = es[0]
    for e in es[1:]:
        tot = tot + e
    for k in range(_K):
        p_ref[:, k:k + 1] = (es[k] / tot)[:, None]


def _topk_stage(e_h_c, e_t):
    rows = e_h_c.shape[0]
    return pl.pallas_call(
        _topk_body,
        grid=(rows // _BR,),
        in_specs=[
            pl.BlockSpec((_BR, _DH), lambda i: (i, 0)),
            pl.BlockSpec((_N, _DH), lambda i: (0, 0)),
        ],
        out_specs=[
            pl.BlockSpec((_BR, _KP), lambda i: (i, 0)),
            pl.BlockSpec((_KP, _BR), lambda i: (0, i)),
        ],
        out_shape=[
            jax.ShapeDtypeStruct((rows, _KP), jnp.float32),
            jax.ShapeDtypeStruct((_KP, rows), jnp.int32),
        ],
    )(e_h_c, e_t)


_NW = 32
_CH = 64
_NBUF = 3


def _sc_gather_body(nch, table_hbm, idx_hbm, out_hbm, idx_v, buf0, buf1, buf2,
                    g0, g1, g2, s0, s1, s2):
    bpw = nch * _CH
    wid = lax.axis_index("s") * 2 + lax.axis_index("c")
    base = wid * bpw
    pltpu.sync_copy(idx_hbm.at[pl.ds(base, bpw)], idx_v)
    bufs = (buf0, buf1, buf2)
    gsems = (g0, g1, g2)
    ssems = (s0, s1, s2)
    gd = [None] * _NBUF
    sd = [None] * _NBUF

    def start_gather(c):
        b = c % _NBUF
        if sd[b] is not None:
            sd[b].wait()
            sd[b] = None
        d = pltpu.make_async_copy(
            table_hbm.at[idx_v.at[pl.ds(c * _CH, _CH)]], bufs[b], gsems[b])
        d.start()
        gd[b] = d

    start_gather(0)
    if nch > 1:
        start_gather(1)
    for c in range(nch):
        b = c % _NBUF
        gd[b].wait()
        d = pltpu.make_async_copy(
            bufs[b], out_hbm.at[pl.ds(base + c * _CH, _CH)], ssems[b])
        d.start()
        sd[b] = d
        if c + 2 < nch:
            start_gather(c + 2)
    for b in range(_NBUF):
        if sd[b] is not None:
            sd[b].wait()


def _sc_gather(table, idx_flat):
    nrows = idx_flat.shape[0]
    bpw = nrows // _NW
    nch = bpw // _CH
    mesh = plsc.VectorSubcoreMesh(core_axis_name="c", subcore_axis_name="s")
    fn = functools.partial(
        pl.kernel,
        mesh=mesh,
        out_type=jax.ShapeDtypeStruct((nrows, _DH), jnp.float32),
        scratch_types=[
            pltpu.VMEM((bpw,), jnp.int32),
            pltpu.VMEM((_CH, _DH), jnp.float32),
            pltpu.VMEM((_CH, _DH), jnp.float32),
            pltpu.VMEM((_CH, _DH), jnp.float32),
            pltpu.SemaphoreType.DMA,
            pltpu.SemaphoreType.DMA,
            pltpu.SemaphoreType.DMA,
            pltpu.SemaphoreType.DMA,
            pltpu.SemaphoreType.DMA,
            pltpu.SemaphoreType.DMA,
        ],
    )(functools.partial(_sc_gather_body, nch))
    return fn(table, idx_flat)


def _agg_body(eh_ref, p_ref, nb_ref, l1w_ref, l1b_ref, l2w_ref, l2b_ref,
              a1w_ref, a1b_ref, a2w_ref, a2b_ref, h_ref, gs_ref):
    eh = eh_ref[...]
    kas = []
    for k in range(_K):
        nbk = nb_ref[k]
        pk = p_ref[:, k:k + 1]
        ehr = pk * nbk + (1.0 - pk) * eh
        gate = jnp.tanh(eh + ehr)
        ka = (jnp.sum(nbk, axis=1, keepdims=True)
              * jnp.sum(gate, axis=1, keepdims=True))
        kas.append(ka)
    mka = kas[0]
    for ka in kas[1:]:
        mka = jnp.maximum(mka, ka)
    eks = [jnp.exp(ka - mka) for ka in kas]
    tot = eks[0]
    for e in eks[1:]:
        tot = tot + e
    e_nh = (eks[0] / tot) * nb_ref[0]
    for k in range(1, _K):
        e_nh = e_nh + (eks[k] / tot) * nb_ref[k]
    sum_emb = jnp.dot(eh + e_nh, l1w_ref[...],
                      preferred_element_type=jnp.float32) + l1b_ref[...]
    sum_emb = jnp.where(sum_emb >= 0, sum_emb, 0.01 * sum_emb)
    bi_emb = jnp.dot(eh * e_nh, l2w_ref[...],
                     preferred_element_type=jnp.float32) + l2b_ref[...]
    bi_emb = jnp.where(bi_emb >= 0, bi_emb, 0.01 * bi_emb)
    h = sum_emb + bi_emb
    h_ref[...] = h
    g1 = jnp.dot(h, a1w_ref[...], preferred_element_type=jnp.float32) + a1b_ref[...]
    g1 = jnp.where(g1 >= 0, g1, 0.01 * g1)
    gs = jnp.sum(g1 * a2w_ref[...], axis=1, keepdims=True) + a2b_ref[...]
    gs_ref[:, 0:1] = gs


def _agg_stage(e_h_c, p, nb3, l1_W, l1b2, l2_W, l2b2, att1_W, a1b2, a2wr, a2b2):
    rows = e_h_c.shape[0]
    return pl.pallas_call(
        _agg_body,
        grid=(rows // _BR,),
        in_specs=[
            pl.BlockSpec((_BR, _DH), lambda i: (i, 0)),
            pl.BlockSpec((_BR, _KP), lambda i: (i, 0)),
            pl.BlockSpec((_K, _BR, _DH), lambda i: (0, i, 0)),
            pl.BlockSpec((_DH, _DH), lambda i: (0, 0)),
            pl.BlockSpec((1, _DH), lambda i: (0, 0)),
            pl.BlockSpec((_DH, _DH), lambda i: (0, 0)),
            pl.BlockSpec((1, _DH), lambda i: (0, 0)),
            pl.BlockSpec((_DH, _DH // 2), lambda i: (0, 0)),
            pl.BlockSpec((1, _DH // 2), lambda i: (0, 0)),
            pl.BlockSpec((1, _DH // 2), lambda i: (0, 0)),
            pl.BlockSpec((1, 1), lambda i: (0, 0)),
        ],
        out_specs=[
            pl.BlockSpec((_BR, _DH), lambda i: (i, 0)),
            pl.BlockSpec((_BR, _KP), lambda i: (i, 0)),
        ],
        out_shape=[
            jax.ShapeDtypeStruct((rows, _DH), jnp.float32),
            jax.ShapeDtypeStruct((rows, _KP), jnp.float32),
        ],
    )(e_h_c, p, nb3, l1_W, l1b2, l2_W, l2b2, att1_W, a1b2, a2wr, a2b2)


def _readout_body(g_ref, b_ref, fcwt_ref, fcb_ref, *rest):
    h_refs = rest[:_NSPLIT]
    gs_refs = rest[_NSPLIT:2 * _NSPLIT]
    lg_ref, yp_ref = rest[2 * _NSPLIT], rest[2 * _NSPLIT + 1]
    gss = [r[...][:, 0:1] for r in gs_refs]
    m = jnp.max(gss[0], axis=0, keepdims=True)
    for g in gss[1:]:
        m = jnp.maximum(m, jnp.max(g, axis=0, keepdims=True))
    es = [jnp.exp(g - m) for g in gss]
    tot = jnp.sum(es[0], axis=0, keepdims=True)
    for e in es[1:]:
        tot = tot + jnp.sum(e, axis=0, keepdims=True)
    hp = jnp.sum(es[0] * h_refs[0][...], axis=0, keepdims=True)
    for e, hr in zip(es[1:], h_refs[1:]):
        hp = hp + jnp.sum(e * hr[...], axis=0, keepdims=True)
    hp = hp / tot
    mu = jnp.mean(hp, axis=1, keepdims=True)
    var = jnp.mean((hp - mu) ** 2, axis=1, keepdims=True)
    hn = (hp - mu) / jnp.sqrt(var + 1e-5) * g_ref[...] + b_ref[...]
    lgs = []
    for c in range(_NCLS):
        v = jnp.sum(hn * fcwt_ref[c:c + 1, :], axis=1, keepdims=True)
        lgs.append(v + fcb_ref[:, c:c + 1])
    m2 = jnp.maximum(lgs[0], lgs[1])
    e2 = [jnp.exp(v - m2) for v in lgs]
    s2 = e2[0] + e2[1]
    for c in range(_NCLS):
        lg_ref[:, c:c + 1] = lgs[c]
        yp_ref[:, c:c + 1] = e2[c] / s2


def _readout_stage(hs, gss, ln_g2, ln_b2, fc_WT, fc_b2):
    return pl.pallas_call(
        _readout_body,
        out_shape=[
            jax.ShapeDtypeStruct((1, _NCLS), jnp.float32),
            jax.ShapeDtypeStruct((1, _NCLS), jnp.float32),
        ],
    )(ln_g2, ln_b2, fc_WT, fc_b2, *hs, *gss)


def kernel(x_s, fc1_W, fc1_b, Wh_W, Wh_b, Wt_W, Wt_b, l1_W, l1_b, l2_W, l2_b,
           att1_W, att1_b, att2_W, att2_b, ln_g, ln_b, fc_W, fc_b):
    fc1_b2 = fc1_b.reshape(1, _DH)
    bh2 = Wh_b.reshape(1, _DH)
    bt2 = Wt_b.reshape(1, _DH)
    l1b2 = l1_b.reshape(1, _DH)
    l2b2 = l2_b.reshape(1, _DH)
    a1b2 = att1_b.reshape(1, _DH // 2)
    a2wr = att2_W.reshape(1, _DH // 2)
    a2b2 = att2_b.reshape(1, 1)
    ln_g2 = ln_g.reshape(1, _DH)
    ln_b2 = ln_b.reshape(1, _DH)
    fc_WT = fc_W.T
    fc_b2 = fc_b.reshape(1, _NCLS)

    x0, csum = _fc1_stage(x_s, fc1_W, fc1_b2)
    e_h, e_t = _proj_stage(x0, csum, Wh_W, bh2, Wt_W, bt2)

    rows = _N // _NSPLIT
    ehcs, pcs, nbcs = [], [], []
    for c in range(_NSPLIT):
        ehc = lax.slice(e_h, (c * rows, 0), ((c + 1) * rows, _DH))
        p_c, idx_c = _topk_stage(ehc, e_t)
        idx_flat = idx_c[:_K].reshape(_K * rows)
        nb_c = _sc_gather(e_t, idx_flat)
        ehcs.append(ehc)
        pcs.append(p_c)
        nbcs.append(nb_c.reshape(_K, rows, _DH))
    hs, gss = [], []
    for c in range(_NSPLIT):
        h_c, gs_c = _agg_stage(ehcs[c], pcs[c], nbcs[c], l1_W, l1b2, l2_W,
                               l2b2, att1_W, a1b2, a2wr, a2b2)
        hs.append(h_c)
        gss.append(gs_c)
    logits, y_prob = _readout_stage(hs, gss, ln_g2, ln_b2, fc_WT, fc_b2)
    return logits, y_prob
